# Initial kernel scaffold; baseline (speedup 1.0000x reference)
import functools

import jax
import jax.numpy as jnp
from jax import lax
from jax.experimental import pallas as pl
from jax.experimental.pallas import tpu as pltpu
from jax.experimental.pallas import tpu_sc as plsc

N = 10000
E = 320000
D = 128
F = 16

NC = 2
NS = 16
NW = NC * NS
B = 128
S = -(-E // (NW * B))
EP = NW * S * B
PAD = EP - E
RPT = 632
NR = NS * RPT
SINK = NR - N

_mesh = plsc.VectorSubcoreMesh(
    core_axis_name="c", subcore_axis_name="s", num_cores=NC, num_subcores=NS)


@functools.partial(
    pl.kernel,
    out_type=jax.ShapeDtypeStruct((NC * NR, F), jnp.float32),
    mesh=_mesh,
    scratch_types=[
        pltpu.VMEM((S, B), jnp.int32),
        pltpu.VMEM((B, F), jnp.float32),
        pltpu.VMEM((RPT, F), jnp.float32),
        pltpu.VMEM_SHARED((NR, F), jnp.float32),
    ],
)
def _deg_pass(dst_hbm, ones_hbm, zeros_hbm, out_hbm, idx_v, ones_v, stage_v,
              acc_sh):
    c = lax.axis_index("c")
    s = lax.axis_index("s")
    wid = s * NC + c
    r0 = s * RPT
    pltpu.sync_copy(zeros_hbm.at[pl.ds(r0, RPT)], stage_v)
    pltpu.sync_copy(stage_v, acc_sh.at[pl.ds(r0, RPT)])
    pltpu.sync_copy(ones_hbm, ones_v)
    pltpu.sync_copy(dst_hbm.at[wid], idx_v)
    plsc.subcore_barrier()

    def step(j, carry):
        pltpu.sync_copy(ones_v, acc_sh.at[idx_v.at[j]], add=True)
        return carry

    lax.fori_loop(0, S, step, 0)
    plsc.subcore_barrier()
    pltpu.sync_copy(acc_sh.at[pl.ds(r0, RPT)], stage_v)
    pltpu.sync_copy(stage_v, out_hbm.at[pl.ds(c * NR + r0, RPT)])


@functools.partial(
    pl.kernel,
    out_type=jax.ShapeDtypeStruct((NC * NR, F), jnp.float32),
    mesh=_mesh,
    scratch_types=[
        pltpu.VMEM((S, B), jnp.int32),
        pltpu.VMEM((S, B), jnp.int32),
        pltpu.VMEM((B, F), jnp.float32),
        pltpu.VMEM((RPT, F), jnp.float32),
        pltpu.VMEM_SHARED((NR, F), jnp.float32),
        pltpu.SemaphoreType.DMA,
    ],
)
def _edge_pass(g_hbm, src_hbm, dst_hbm, zeros_hbm, out_hbm, sidx_v, didx_v,
               rows_v, stage_v, acc_sh, sem):
    c = lax.axis_index("c")
    s = lax.axis_index("s")
    wid = s * NC + c
    r0 = s * RPT
    pltpu.sync_copy(zeros_hbm.at[pl.ds(r0, RPT)], stage_v)
    pltpu.sync_copy(stage_v, acc_sh.at[pl.ds(r0, RPT)])
    pltpu.sync_copy(src_hbm.at[wid], sidx_v)
    pltpu.sync_copy(dst_hbm.at[wid], didx_v)
    plsc.subcore_barrier()

    def step(j, carry):
        pltpu.async_copy(g_hbm.at[sidx_v.at[j]], rows_v, sem).wait()
        pltpu.sync_copy(rows_v, acc_sh.at[didx_v.at[j]], add=True)
        return carry

    lax.fori_loop(0, S, step, 0)
    plsc.subcore_barrier()
    pltpu.sync_copy(acc_sh.at[pl.ds(r0, RPT)], stage_v)
    pltpu.sync_copy(stage_v, out_hbm.at[pl.ds(c * NR + r0, RPT)])


_MM1_ROWS = 2000


def _mm1_body(x_ref, w_ref, o_ref):
    o_ref[...] = jnp.dot(x_ref[...], w_ref[...],
                         preferred_element_type=jnp.float32)


_mm1 = pl.pallas_call(
    _mm1_body,
    grid=(N // _MM1_ROWS,),
    in_specs=[
        pl.BlockSpec((_MM1_ROWS, D), lambda i: (i, 0)),
        pl.BlockSpec((D, F), lambda i: (0, 0)),
    ],
    out_specs=pl.BlockSpec((_MM1_ROWS, F), lambda i: (i, 0)),
    out_shape=jax.ShapeDtypeStruct((N, F), jnp.float32),
)


def _scale_body(dpa_ref, dpb_ref, h1_ref, dis_ref, g_ref):
    deg = dpa_ref[...] + dpb_ref[...] + 1.0
    dis = lax.rsqrt(deg)
    dis_ref[...] = dis
    g_ref[...] = h1_ref[...] * dis


_scale = pl.pallas_call(
    _scale_body,
    out_shape=(jax.ShapeDtypeStruct((N, F), jnp.float32),
               jax.ShapeDtypeStruct((N, F), jnp.float32)),
)


def _mid_body(p0_ref, p1_ref, g1_ref, dis_ref, w2_ref, b1_ref, g2_ref):
    dis = dis_ref[...]
    a = dis * (g1_ref[...] + p0_ref[...] + p1_ref[...]) + b1_ref[...]
    a = jnp.maximum(a, 0.0)
    h2 = jnp.dot(a, w2_ref[...], preferred_element_type=jnp.float32)
    g2_ref[...] = h2 * dis


_mid = pl.pallas_call(
    _mid_body,
    out_shape=jax.ShapeDtypeStruct((N, F), jnp.float32),
)


def _out_body(q0_ref, q1_ref, g2_ref, dis_ref, b2_ref, o_ref):
    z = dis_ref[...] * (g2_ref[...] + q0_ref[...] + q1_ref[...]) + b2_ref[...]
    m = jnp.max(z, axis=1, keepdims=True)
    e = jnp.exp(z - m)
    o_ref[...] = z - m - jnp.log(jnp.sum(e, axis=1, keepdims=True))


_outk = pl.pallas_call(
    _out_body,
    out_shape=jax.ShapeDtypeStruct((N, F), jnp.float32),
)


def kernel(x, edge_index, W1, b1, W2, b2):
    ei = edge_index.astype(jnp.int32)
    pad = jnp.arange(PAD, dtype=jnp.int32)
    src_p = jnp.concatenate([ei[0], pad % 128]).reshape(NW, S, B)
    dst_p = jnp.concatenate([ei[1], N + (pad % SINK)]).reshape(NW, S, B)
    ones2d = jnp.ones((B, F), jnp.float32)
    zeros2d = jnp.zeros((NR, F), jnp.float32)

    degp = _deg_pass(dst_p, ones2d, zeros2d)
    h1 = _mm1(x, W1)
    dis2, g1 = _scale(degp[0:N], degp[NR:NR + N], h1)
    part1 = _edge_pass(g1, src_p, dst_p, zeros2d)
    g2 = _mid(part1[0:N], part1[NR:NR + N], g1, dis2, W2, b1)
    part2 = _edge_pass(g2, src_p, dst_p, zeros2d)
    return _outk(part2[0:N], part2[NR:NR + N], g2, dis2, b2)


# speedup vs baseline: 32.6308x; 32.6308x over previous
import functools

import jax
import jax.numpy as jnp
from jax import lax
from jax.experimental import pallas as pl
from jax.experimental.pallas import tpu as pltpu
from jax.experimental.pallas import tpu_sc as plsc

N = 10000
E = 320000
D = 128
F = 16

NC = 2
NS = 16
NW = NC * NS
B = 128
S = -(-E // (NW * B))
EP = NW * S * B
PAD = EP - E
RPT = 632
NR = NS * RPT
SINK = NR - N

_mesh = plsc.VectorSubcoreMesh(
    core_axis_name="c", subcore_axis_name="s", num_cores=NC, num_subcores=NS)
_sc_params = pltpu.CompilerParams(use_tc_tiling_on_sc=False)


@functools.partial(
    pl.kernel,
    out_type=jax.ShapeDtypeStruct((NC * NR, F), jnp.float32),
    mesh=_mesh,
    scratch_types=[
        pltpu.VMEM((S, B), jnp.int32),
        pltpu.VMEM((B, F), jnp.float32),
        pltpu.VMEM((RPT, F), jnp.float32),
        pltpu.VMEM_SHARED((NR, F), jnp.float32),
    ],
    compiler_params=_sc_params,
)
def _deg_pass(dst_hbm, ones_hbm, zeros_hbm, out_hbm, idx_v, ones_v, stage_v,
              acc_sh):
    c = lax.axis_index("c")
    s = lax.axis_index("s")
    wid = s * NC + c
    r0 = s * RPT
    pltpu.sync_copy(zeros_hbm.at[pl.ds(r0, RPT)], stage_v)
    pltpu.sync_copy(stage_v, acc_sh.at[pl.ds(r0, RPT)])
    pltpu.sync_copy(ones_hbm, ones_v)
    pltpu.sync_copy(dst_hbm.at[wid], idx_v)
    plsc.subcore_barrier()

    def step(j, carry):
        pltpu.sync_copy(ones_v, acc_sh.at[idx_v.at[j]], add=True)
        return carry

    lax.fori_loop(0, S, step, 0)
    plsc.subcore_barrier()
    pltpu.sync_copy(acc_sh.at[pl.ds(r0, RPT)], stage_v)
    pltpu.sync_copy(stage_v, out_hbm.at[pl.ds(c * NR + r0, RPT)])


@functools.partial(
    pl.kernel,
    out_type=jax.ShapeDtypeStruct((NC * NR, F), jnp.float32),
    mesh=_mesh,
    scratch_types=[
        pltpu.VMEM((S, B), jnp.int32),
        pltpu.VMEM((S, B), jnp.int32),
        pltpu.VMEM((B, F), jnp.float32),
        pltpu.VMEM((RPT, F), jnp.float32),
        pltpu.VMEM_SHARED((NR, F), jnp.float32),
        pltpu.SemaphoreType.DMA,
    ],
    compiler_params=_sc_params,
)
def _edge_pass(g_hbm, src_hbm, dst_hbm, zeros_hbm, out_hbm, sidx_v, didx_v,
               rows_v, stage_v, acc_sh, sem):
    c = lax.axis_index("c")
    s = lax.axis_index("s")
    wid = s * NC + c
    r0 = s * RPT
    pltpu.sync_copy(zeros_hbm.at[pl.ds(r0, RPT)], stage_v)
    pltpu.sync_copy(stage_v, acc_sh.at[pl.ds(r0, RPT)])
    pltpu.sync_copy(src_hbm.at[wid], sidx_v)
    pltpu.sync_copy(dst_hbm.at[wid], didx_v)
    plsc.subcore_barrier()

    def step(j, carry):
        pltpu.async_copy(g_hbm.at[sidx_v.at[j]], rows_v, sem).wait()
        pltpu.sync_copy(rows_v, acc_sh.at[didx_v.at[j]], add=True)
        return carry

    lax.fori_loop(0, S, step, 0)
    plsc.subcore_barrier()
    pltpu.sync_copy(acc_sh.at[pl.ds(r0, RPT)], stage_v)
    pltpu.sync_copy(stage_v, out_hbm.at[pl.ds(c * NR + r0, RPT)])


_MM1_ROWS = 2000


def _mm1_body(x_ref, w_ref, o_ref):
    o_ref[...] = jnp.dot(x_ref[...], w_ref[...],
                         preferred_element_type=jnp.float32)


_mm1 = pl.pallas_call(
    _mm1_body,
    grid=(N // _MM1_ROWS,),
    in_specs=[
        pl.BlockSpec((_MM1_ROWS, D), lambda i: (i, 0)),
        pl.BlockSpec((D, F), lambda i: (0, 0)),
    ],
    out_specs=pl.BlockSpec((_MM1_ROWS, F), lambda i: (i, 0)),
    out_shape=jax.ShapeDtypeStruct((N, F), jnp.float32),
)


def _scale_body(dpa_ref, dpb_ref, h1_ref, dis_ref, g_ref):
    deg = dpa_ref[...] + dpb_ref[...] + 1.0
    dis = lax.rsqrt(deg)
    dis_ref[...] = dis
    g_ref[...] = h1_ref[...] * dis


_scale = pl.pallas_call(
    _scale_body,
    out_shape=(jax.ShapeDtypeStruct((N, F), jnp.float32),
               jax.ShapeDtypeStruct((N, F), jnp.float32)),
)


def _mid_body(p0_ref, p1_ref, g1_ref, dis_ref, w2_ref, b1_ref, g2_ref):
    dis = dis_ref[...]
    a = dis * (g1_ref[...] + p0_ref[...] + p1_ref[...]) + b1_ref[...]
    a = jnp.maximum(a, 0.0)
    h2 = jnp.dot(a, w2_ref[...], preferred_element_type=jnp.float32)
    g2_ref[...] = h2 * dis


_mid = pl.pallas_call(
    _mid_body,
    out_shape=jax.ShapeDtypeStruct((N, F), jnp.float32),
)


def _out_body(q0_ref, q1_ref, g2_ref, dis_ref, b2_ref, o_ref):
    z = dis_ref[...] * (g2_ref[...] + q0_ref[...] + q1_ref[...]) + b2_ref[...]
    m = jnp.max(z, axis=1, keepdims=True)
    e = jnp.exp(z - m)
    o_ref[...] = z - m - jnp.log(jnp.sum(e, axis=1, keepdims=True))


_outk = pl.pallas_call(
    _out_body,
    out_shape=jax.ShapeDtypeStruct((N, F), jnp.float32),
)


def kernel(x, edge_index, W1, b1, W2, b2):
    ei = edge_index.astype(jnp.int32)
    pad = jnp.arange(PAD, dtype=jnp.int32)
    src_p = jnp.concatenate([ei[0], pad % 128]).reshape(NW, S, B)
    dst_p = jnp.concatenate([ei[1], N + (pad % SINK)]).reshape(NW, S, B)
    ones2d = jnp.ones((B, F), jnp.float32)
    zeros2d = jnp.zeros((NR, F), jnp.float32)

    degp = _deg_pass(dst_p, ones2d, zeros2d)
    h1 = _mm1(x, W1)
    dis2, g1 = _scale(degp[0:N], degp[NR:NR + N], h1)
    part1 = _edge_pass(g1, src_p, dst_p, zeros2d)
    g2 = _mid(part1[0:N], part1[NR:NR + N], g1, dis2, W2, b1)
    part2 = _edge_pass(g2, src_p, dst_p, zeros2d)
    return _outk(part2[0:N], part2[NR:NR + N], g2, dis2, b2)


# speedup vs baseline: 52.0626x; 1.5955x over previous
import functools

import jax
import jax.numpy as jnp
from jax import lax
from jax.experimental import pallas as pl
from jax.experimental.pallas import tpu as pltpu
from jax.experimental.pallas import tpu_sc as plsc

N = 10000
E = 320000
D = 128
F = 16

NC = 2
NS = 16
NW = NC * NS
B = 128
NB = 8
S = 80
G = S // NB
EP = NW * S * B
PAD = EP - E
RPT = 632
NR = NS * RPT
SINK = NR - N

_mesh = plsc.VectorSubcoreMesh(
    core_axis_name="c", subcore_axis_name="s", num_cores=NC, num_subcores=NS)
_sc_params = pltpu.CompilerParams(use_tc_tiling_on_sc=False)


@functools.partial(
    pl.kernel,
    out_type=jax.ShapeDtypeStruct((NC * NR, F), jnp.float32),
    mesh=_mesh,
    scratch_types=[
        pltpu.VMEM((S, B), jnp.int32),
        pltpu.VMEM((B, F), jnp.float32),
        pltpu.VMEM((RPT, F), jnp.float32),
        pltpu.VMEM_SHARED((NR, F), jnp.float32),
        pltpu.SemaphoreType.DMA,
    ],
    compiler_params=_sc_params,
)
def _deg_pass(dst_hbm, ones_hbm, zeros_hbm, out_hbm, idx_v, ones_v, stage_v,
              acc_sh, sem):
    c = lax.axis_index("c")
    s = lax.axis_index("s")
    wid = s * NC + c
    r0 = s * RPT
    pltpu.sync_copy(zeros_hbm.at[pl.ds(r0, RPT)], stage_v)
    pltpu.sync_copy(stage_v, acc_sh.at[pl.ds(r0, RPT)])
    pltpu.sync_copy(ones_hbm, ones_v)
    pltpu.sync_copy(dst_hbm.at[wid], idx_v)
    plsc.subcore_barrier()

    def step(j, carry):
        pltpu.async_copy(ones_v, acc_sh.at[idx_v.at[j]], sem, add=True)
        return carry

    lax.fori_loop(0, S, step, 0)

    def drain(j, carry):
        pltpu.make_async_copy(ones_v, acc_sh.at[idx_v.at[0]], sem).wait()
        return carry

    lax.fori_loop(0, S, drain, 0)
    plsc.subcore_barrier()
    pltpu.sync_copy(acc_sh.at[pl.ds(r0, RPT)], stage_v)
    pltpu.sync_copy(stage_v, out_hbm.at[pl.ds(c * NR + r0, RPT)])


@functools.partial(
    pl.kernel,
    out_type=jax.ShapeDtypeStruct((NC * NR, F), jnp.float32),
    mesh=_mesh,
    scratch_types=[
        pltpu.VMEM((S, B), jnp.int32),
        pltpu.VMEM((S, B), jnp.int32),
        pltpu.VMEM((NB, B, F), jnp.float32),
        pltpu.VMEM((RPT, F), jnp.float32),
        pltpu.VMEM_SHARED((NR, F), jnp.float32),
    ] + [pltpu.SemaphoreType.DMA] * (2 * NB),
    compiler_params=_sc_params,
)
def _edge_pass(g_hbm, src_hbm, dst_hbm, zeros_hbm, out_hbm, sidx_v, didx_v,
               rows_v, stage_v, acc_sh, *sems):
    c = lax.axis_index("c")
    s = lax.axis_index("s")
    wid = s * NC + c
    r0 = s * RPT
    pltpu.sync_copy(zeros_hbm.at[pl.ds(r0, RPT)], stage_v)
    pltpu.sync_copy(stage_v, acc_sh.at[pl.ds(r0, RPT)])
    pltpu.sync_copy(src_hbm.at[wid], sidx_v)
    pltpu.sync_copy(dst_hbm.at[wid], didx_v)
    plsc.subcore_barrier()

    for b in range(NB):
        pltpu.async_copy(g_hbm.at[sidx_v.at[b]], rows_v.at[b], sems[b])

    def round_body(g, carry):
        jb = g * NB
        for b in range(NB):
            pltpu.make_async_copy(
                g_hbm.at[sidx_v.at[0]], rows_v.at[b], sems[b]).wait()
            pltpu.async_copy(
                rows_v.at[b], acc_sh.at[didx_v.at[jb + b]], sems[NB + b],
                add=True)
        for b in range(NB):
            pltpu.make_async_copy(
                rows_v.at[b], acc_sh.at[didx_v.at[0]], sems[NB + b]).wait()

            @pl.when(g < G - 1)
            def _():
                pltpu.async_copy(
                    g_hbm.at[sidx_v.at[jb + NB + b]], rows_v.at[b], sems[b])

        return carry

    lax.fori_loop(0, G, round_body, 0)
    plsc.subcore_barrier()
    pltpu.sync_copy(acc_sh.at[pl.ds(r0, RPT)], stage_v)
    pltpu.sync_copy(stage_v, out_hbm.at[pl.ds(c * NR + r0, RPT)])


_MM1_ROWS = 2000


def _mm1_body(x_ref, w_ref, o_ref):
    o_ref[...] = jnp.dot(x_ref[...], w_ref[...],
                         preferred_element_type=jnp.float32)


_mm1 = pl.pallas_call(
    _mm1_body,
    grid=(N // _MM1_ROWS,),
    in_specs=[
        pl.BlockSpec((_MM1_ROWS, D), lambda i: (i, 0)),
        pl.BlockSpec((D, F), lambda i: (0, 0)),
    ],
    out_specs=pl.BlockSpec((_MM1_ROWS, F), lambda i: (i, 0)),
    out_shape=jax.ShapeDtypeStruct((N, F), jnp.float32),
)


def _scale_body(dpa_ref, dpb_ref, h1_ref, dis_ref, g_ref):
    deg = dpa_ref[...] + dpb_ref[...] + 1.0
    dis = lax.rsqrt(deg)
    dis_ref[...] = dis
    g_ref[...] = h1_ref[...] * dis


_scale = pl.pallas_call(
    _scale_body,
    out_shape=(jax.ShapeDtypeStruct((N, F), jnp.float32),
               jax.ShapeDtypeStruct((N, F), jnp.float32)),
)


def _mid_body(p0_ref, p1_ref, g1_ref, dis_ref, w2_ref, b1_ref, g2_ref):
    dis = dis_ref[...]
    a = dis * (g1_ref[...] + p0_ref[...] + p1_ref[...]) + b1_ref[...]
    a = jnp.maximum(a, 0.0)
    h2 = jnp.dot(a, w2_ref[...], preferred_element_type=jnp.float32)
    g2_ref[...] = h2 * dis


_mid = pl.pallas_call(
    _mid_body,
    out_shape=jax.ShapeDtypeStruct((N, F), jnp.float32),
)


def _out_body(q0_ref, q1_ref, g2_ref, dis_ref, b2_ref, o_ref):
    z = dis_ref[...] * (g2_ref[...] + q0_ref[...] + q1_ref[...]) + b2_ref[...]
    m = jnp.max(z, axis=1, keepdims=True)
    e = jnp.exp(z - m)
    o_ref[...] = z - m - jnp.log(jnp.sum(e, axis=1, keepdims=True))


_outk = pl.pallas_call(
    _out_body,
    out_shape=jax.ShapeDtypeStruct((N, F), jnp.float32),
)


def kernel(x, edge_index, W1, b1, W2, b2):
    ei = edge_index.astype(jnp.int32)
    pad = jnp.arange(PAD, dtype=jnp.int32)
    src_p = jnp.concatenate([ei[0], pad % 128]).reshape(NW, S, B)
    dst_p = jnp.concatenate([ei[1], N + (pad % SINK)]).reshape(NW, S, B)
    ones2d = jnp.ones((B, F), jnp.float32)
    zeros2d = jnp.zeros((NR, F), jnp.float32)

    degp = _deg_pass(dst_p, ones2d, zeros2d)
    h1 = _mm1(x, W1)
    dis2, g1 = _scale(degp[0:N], degp[NR:NR + N], h1)
    part1 = _edge_pass(g1, src_p, dst_p, zeros2d)
    g2 = _mid(part1[0:N], part1[NR:NR + N], g1, dis2, W2, b1)
    part2 = _edge_pass(g2, src_p, dst_p, zeros2d)
    return _outk(part2[0:N], part2[NR:NR + N], g2, dis2, b2)


# speedup vs baseline: 63.2400x; 1.2147x over previous
import functools

import jax
import jax.numpy as jnp
from jax import lax
from jax.experimental import pallas as pl
from jax.experimental.pallas import tpu as pltpu
from jax.experimental.pallas import tpu_sc as plsc

N = 10000
E = 320000
D = 128
F = 16

NC = 2
NS = 16
NW = NC * NS
B = 128
NB = 8
S = 80
G = S // NB
EP = NW * S * B
PAD = EP - E
RPT = 632
NR = NS * RPT
SINK = NR - N

_mesh = plsc.VectorSubcoreMesh(
    core_axis_name="c", subcore_axis_name="s", num_cores=NC, num_subcores=NS)
_sc_params = pltpu.CompilerParams(use_tc_tiling_on_sc=False)


@functools.partial(
    pl.kernel,
    out_type=jax.ShapeDtypeStruct((NC * NR, F), jnp.float32),
    mesh=_mesh,
    scratch_types=[
        pltpu.VMEM((S, B), jnp.int32),
        pltpu.VMEM((B, F), jnp.float32),
        pltpu.VMEM((RPT, F), jnp.float32),
        pltpu.VMEM_SHARED((NR, F), jnp.float32),
        pltpu.SemaphoreType.DMA,
    ],
    compiler_params=_sc_params,
)
def _deg_pass(dst_hbm, ones_hbm, zeros_hbm, out_hbm, idx_v, ones_v, stage_v,
              acc_sh, sem):
    c = lax.axis_index("c")
    s = lax.axis_index("s")
    wid = s * NC + c
    r0 = s * RPT
    pltpu.sync_copy(zeros_hbm.at[pl.ds(r0, RPT)], stage_v)
    pltpu.sync_copy(stage_v, acc_sh.at[pl.ds(r0, RPT)])
    pltpu.sync_copy(ones_hbm, ones_v)
    pltpu.sync_copy(dst_hbm.at[wid], idx_v)
    plsc.subcore_barrier()

    def step(j, carry):
        pltpu.async_copy(ones_v, acc_sh.at[idx_v.at[j]], sem, add=True)
        return carry

    lax.fori_loop(0, S, step, 0)

    def drain(j, carry):
        pltpu.make_async_copy(ones_v, acc_sh.at[idx_v.at[0]], sem).wait()
        return carry

    lax.fori_loop(0, S, drain, 0)
    plsc.subcore_barrier()
    pltpu.sync_copy(acc_sh.at[pl.ds(r0, RPT)], stage_v)
    pltpu.sync_copy(stage_v, out_hbm.at[pl.ds(c * NR + r0, RPT)])


@functools.partial(
    pl.kernel,
    out_type=jax.ShapeDtypeStruct((NC * NR, F), jnp.float32),
    mesh=_mesh,
    scratch_types=[
        pltpu.VMEM((S, B), jnp.int32),
        pltpu.VMEM((S, B), jnp.int32),
        pltpu.VMEM((NB, B, F), jnp.float32),
        pltpu.VMEM((RPT, F), jnp.float32),
        pltpu.VMEM_SHARED((NR, F), jnp.float32),
    ] + [pltpu.SemaphoreType.DMA] * (2 * NB),
    compiler_params=_sc_params,
)
def _edge_pass(g_hbm, src_hbm, dst_hbm, zeros_hbm, out_hbm, sidx_v, didx_v,
               rows_v, stage_v, acc_sh, *sems):
    c = lax.axis_index("c")
    s = lax.axis_index("s")
    wid = s * NC + c
    r0 = s * RPT
    pltpu.sync_copy(zeros_hbm.at[pl.ds(r0, RPT)], stage_v)
    pltpu.sync_copy(stage_v, acc_sh.at[pl.ds(r0, RPT)])
    pltpu.sync_copy(src_hbm.at[wid], sidx_v)
    pltpu.sync_copy(dst_hbm.at[wid], didx_v)
    plsc.subcore_barrier()

    for b in range(NB):
        pltpu.async_copy(g_hbm.at[sidx_v.at[b]], rows_v.at[b], sems[b])

    def round_body(g, carry):
        jb = g * NB
        for b in range(NB):
            pltpu.make_async_copy(
                g_hbm.at[sidx_v.at[0]], rows_v.at[b], sems[b]).wait()
            pltpu.async_copy(
                rows_v.at[b], acc_sh.at[didx_v.at[jb + b]], sems[NB + b],
                add=True)
        for b in range(NB):
            pltpu.make_async_copy(
                rows_v.at[b], acc_sh.at[didx_v.at[0]], sems[NB + b]).wait()

            @pl.when(g < G - 1)
            def _():
                pltpu.async_copy(
                    g_hbm.at[sidx_v.at[jb + NB + b]], rows_v.at[b], sems[b])

        return carry

    lax.fori_loop(0, G, round_body, 0)
    plsc.subcore_barrier()
    pltpu.sync_copy(acc_sh.at[pl.ds(r0, RPT)], stage_v)
    pltpu.sync_copy(stage_v, out_hbm.at[pl.ds(c * NR + r0, RPT)])


NP = N // 8
NR8 = NR // 8
def _mm1_body(x_ref, w_ref, o_ref):
    o_ref[...] = jnp.dot(x_ref[...], w_ref[...],
                         preferred_element_type=jnp.float32)


_mm1 = pl.pallas_call(
    _mm1_body,
    out_shape=jax.ShapeDtypeStruct((NP, 128), jnp.float32),
)


def _scale_body(dpa_ref, dpb_ref, h1_ref, dis_ref, g_ref):
    deg = dpa_ref[...] + dpb_ref[...] + 1.0
    dis = lax.rsqrt(deg)
    dis_ref[...] = dis
    g_ref[...] = h1_ref[...] * dis


_scale = pl.pallas_call(
    _scale_body,
    out_shape=(jax.ShapeDtypeStruct((NP, 128), jnp.float32),
               jax.ShapeDtypeStruct((NP, 128), jnp.float32)),
)


def _mid_body(p0_ref, p1_ref, g1_ref, dis_ref, w2_ref, b1_ref, g2_ref):
    dis = dis_ref[...]
    a = dis * (g1_ref[...] + p0_ref[...] + p1_ref[...]) + b1_ref[...]
    a = jnp.maximum(a, 0.0)
    h2 = jnp.dot(a, w2_ref[...], preferred_element_type=jnp.float32)
    g2_ref[...] = h2 * dis


_mid = pl.pallas_call(
    _mid_body,
    out_shape=jax.ShapeDtypeStruct((NP, 128), jnp.float32),
)


def _out_body(q0_ref, q1_ref, g2_ref, dis_ref, b2_ref, mgrp_ref, o_ref):
    zp = (dis_ref[...] * (g2_ref[...] + q0_ref[...] + q1_ref[...])
          + b2_ref[...])
    m = jnp.max(zp, axis=1, keepdims=True)
    e = jnp.exp(zp - m)
    s = jnp.dot(e, mgrp_ref[...], preferred_element_type=jnp.float32)
    o_ref[...] = zp - m - jnp.log(s)


_outk = pl.pallas_call(
    _out_body,
    out_shape=jax.ShapeDtypeStruct((NP, 128), jnp.float32),
)


def kernel(x, edge_index, W1, b1, W2, b2):
    ei = edge_index.astype(jnp.int32)
    pad = jnp.arange(PAD, dtype=jnp.int32)
    src_p = jnp.concatenate([ei[0], pad % 128]).reshape(NW, S, B)
    dst_p = jnp.concatenate([ei[1], N + (pad % SINK)]).reshape(NW, S, B)
    ones2d = jnp.ones((B, F), jnp.float32)
    zeros2d = jnp.zeros((NR, F), jnp.float32)
    w1blk = jnp.kron(jnp.eye(8, dtype=jnp.float32), W1)
    w2blk = jnp.kron(jnp.eye(8, dtype=jnp.float32), W2)
    b1t = jnp.tile(b1, 8).reshape(1, 128)
    b2t = jnp.tile(b2, 8).reshape(1, 128)

    degp = _deg_pass(dst_p, ones2d, zeros2d).reshape(NC * NR8, 128)
    h1p = _mm1(x.reshape(NP, 8 * D), w1blk)
    dis2p, g1p = _scale(degp[0:NP], degp[NR8:NR8 + NP], h1p)
    part1 = _edge_pass(g1p.reshape(N, F), src_p, dst_p,
                       zeros2d).reshape(NC * NR8, 128)
    g2p = _mid(part1[0:NP], part1[NR8:NR8 + NP], g1p, dis2p, w2blk, b1t)
    part2 = _edge_pass(g2p.reshape(N, F), src_p, dst_p,
                       zeros2d).reshape(NC * NR8, 128)
    mgrp = jnp.kron(jnp.eye(8, dtype=jnp.float32),
                    jnp.ones((F, F), jnp.float32))
    outp = _outk(part2[0:NP], part2[NR8:NR8 + NP], g2p, dis2p, b2t, mgrp)
    return outp.reshape(N, F)


# speedup vs baseline: 86.2932x; 1.3645x over previous
import functools

import jax
import jax.numpy as jnp
from jax import lax
from jax.experimental import pallas as pl
from jax.experimental.pallas import tpu as pltpu
from jax.experimental.pallas import tpu_sc as plsc

N = 10000
E = 320000
D = 128
F = 16

NC = 2
NS = 16
NW = NC * NS
B = 128
NB = 8
S = 80
G = S // NB
EP = NW * S * B
PAD = EP - E
RPT = 632
NR = NS * RPT
SINK = NR - N
NP = N // 8
NR8 = NR // 8
RPT8 = RPT // 8

_mesh = plsc.VectorSubcoreMesh(
    core_axis_name="c", subcore_axis_name="s", num_cores=NC, num_subcores=NS)
_sc_params = pltpu.CompilerParams(use_tc_tiling_on_sc=False)


def _pack_writeout(acc_sh, stage_v, pack_v, out_hbm, c, s):
    r0 = s * RPT
    pltpu.sync_copy(acc_sh.at[pl.ds(r0, RPT)], stage_v)

    def repack(r, carry):
        for k in range(8):
            pack_v[r, pl.ds(16 * k, 16)] = stage_v[r * 8 + k, :]
        return carry

    lax.fori_loop(0, RPT8, repack, 0)
    pltpu.sync_copy(pack_v, out_hbm.at[pl.ds(c * NR8 + s * RPT8, RPT8)])


@functools.partial(
    pl.kernel,
    out_type=jax.ShapeDtypeStruct((NC * NR8, 128), jnp.float32),
    mesh=_mesh,
    scratch_types=[
        pltpu.VMEM((S, B), jnp.int32),
        pltpu.VMEM((B, F), jnp.float32),
        pltpu.VMEM((RPT, F), jnp.float32),
        pltpu.VMEM((RPT8, 128), jnp.float32),
        pltpu.VMEM_SHARED((NR, F), jnp.float32),
        pltpu.SemaphoreType.DMA,
    ],
    compiler_params=_sc_params,
)
def _deg_pass(dst_hbm, ones_hbm, zeros_hbm, out_hbm, idx_v, ones_v, stage_v,
              pack_v, acc_sh, sem):
    c = lax.axis_index("c")
    s = lax.axis_index("s")
    wid = s * NC + c
    r0 = s * RPT
    pltpu.sync_copy(zeros_hbm.at[pl.ds(r0, RPT)], stage_v)
    pltpu.sync_copy(stage_v, acc_sh.at[pl.ds(r0, RPT)])
    pltpu.sync_copy(ones_hbm, ones_v)
    pltpu.sync_copy(dst_hbm.at[wid], idx_v)
    plsc.subcore_barrier()

    def step(j, carry):
        pltpu.async_copy(ones_v, acc_sh.at[idx_v.at[j]], sem, add=True)
        return carry

    lax.fori_loop(0, S, step, 0)

    def drain(j, carry):
        pltpu.make_async_copy(ones_v, acc_sh.at[idx_v.at[0]], sem).wait()
        return carry

    lax.fori_loop(0, S, drain, 0)
    plsc.subcore_barrier()
    _pack_writeout(acc_sh, stage_v, pack_v, out_hbm, c, s)


@functools.partial(
    pl.kernel,
    out_type=jax.ShapeDtypeStruct((NC * NR8, 128), jnp.float32),
    mesh=_mesh,
    scratch_types=[
        pltpu.VMEM((S, B), jnp.int32),
        pltpu.VMEM((S, B), jnp.int32),
        pltpu.VMEM((NB, B, F), jnp.float32),
        pltpu.VMEM((RPT, F), jnp.float32),
        pltpu.VMEM((RPT8, 128), jnp.float32),
        pltpu.VMEM_SHARED((NR, F), jnp.float32),
    ] + [pltpu.SemaphoreType.DMA] * (2 * NB),
    compiler_params=_sc_params,
)
def _edge_pass(g_hbm, src_hbm, dst_hbm, zeros_hbm, out_hbm, sidx_v, didx_v,
               rows_v, stage_v, pack_v, acc_sh, *sems):
    c = lax.axis_index("c")
    s = lax.axis_index("s")
    wid = s * NC + c
    r0 = s * RPT
    pltpu.sync_copy(zeros_hbm.at[pl.ds(r0, RPT)], stage_v)
    pltpu.sync_copy(stage_v, acc_sh.at[pl.ds(r0, RPT)])
    pltpu.sync_copy(src_hbm.at[wid], sidx_v)
    pltpu.sync_copy(dst_hbm.at[wid], didx_v)
    plsc.subcore_barrier()

    for b in range(NB):
        pltpu.async_copy(g_hbm.at[sidx_v.at[b]], rows_v.at[b], sems[b])

    def round_body(g, carry):
        jb = g * NB
        for b in range(NB):
            pltpu.make_async_copy(
                g_hbm.at[sidx_v.at[0]], rows_v.at[b], sems[b]).wait()
            pltpu.async_copy(
                rows_v.at[b], acc_sh.at[didx_v.at[jb + b]], sems[NB + b],
                add=True)
        for b in range(NB):
            pltpu.make_async_copy(
                rows_v.at[b], acc_sh.at[didx_v.at[0]], sems[NB + b]).wait()

            @pl.when(g < G - 1)
            def _():
                pltpu.async_copy(
                    g_hbm.at[sidx_v.at[jb + NB + b]], rows_v.at[b], sems[b])

        return carry

    lax.fori_loop(0, G, round_body, 0)
    plsc.subcore_barrier()
    _pack_writeout(acc_sh, stage_v, pack_v, out_hbm, c, s)


_CH = 20480
_PGRID = EP // _CH


def _prep_body(e_ref, s_ref, d_ref):
    i = pl.program_id(0)
    col = i * _CH + lax.broadcasted_iota(jnp.int32, (_CH,), 0)
    real = col < E
    s_ref[...] = jnp.where(real, e_ref[0, :], col % 128)
    d_ref[...] = jnp.where(real, e_ref[1, :], N + col % SINK)


_prep = pl.pallas_call(
    _prep_body,
    grid=(_PGRID,),
    in_specs=[pl.BlockSpec((2, _CH), lambda i: (0, i))],
    out_specs=(pl.BlockSpec((_CH,), lambda i: (i,)),
               pl.BlockSpec((_CH,), lambda i: (i,))),
    out_shape=(jax.ShapeDtypeStruct((EP,), jnp.int32),
               jax.ShapeDtypeStruct((EP,), jnp.int32)),
)


def _mm1_body(x_ref, w_ref, o_ref):
    o_ref[...] = jnp.dot(x_ref[...], w_ref[...],
                         preferred_element_type=jnp.float32)


_mm1 = pl.pallas_call(
    _mm1_body,
    out_shape=jax.ShapeDtypeStruct((NP, 128), jnp.float32),
)


def _scale_body(degp_ref, h1_ref, dis_ref, g_ref):
    deg = degp_ref[0:NP] + degp_ref[NR8:NR8 + NP] + 1.0
    dis = lax.rsqrt(deg)
    dis_ref[...] = dis
    g_ref[...] = h1_ref[...] * dis


_scale = pl.pallas_call(
    _scale_body,
    out_shape=(jax.ShapeDtypeStruct((NP, 128), jnp.float32),
               jax.ShapeDtypeStruct((NP, 128), jnp.float32)),
)


def _mid_body(part_ref, g1_ref, dis_ref, w2_ref, b1_ref, g2_ref):
    dis = dis_ref[...]
    a = (dis * (g1_ref[...] + part_ref[0:NP] + part_ref[NR8:NR8 + NP])
         + b1_ref[...])
    a = jnp.maximum(a, 0.0)
    h2 = jnp.dot(a, w2_ref[...], preferred_element_type=jnp.float32)
    g2_ref[...] = h2 * dis


_mid = pl.pallas_call(
    _mid_body,
    out_shape=jax.ShapeDtypeStruct((NP, 128), jnp.float32),
)


def _out_body(part_ref, g2_ref, dis_ref, b2_ref, mgrp_ref, o_ref):
    zp = (dis_ref[...] * (g2_ref[...] + part_ref[0:NP]
                          + part_ref[NR8:NR8 + NP]) + b2_ref[...])
    m = jnp.max(zp, axis=1, keepdims=True)
    e = jnp.exp(zp - m)
    s = jnp.dot(e, mgrp_ref[...], preferred_element_type=jnp.float32)
    o_ref[...] = zp - m - jnp.log(s)


_outk = pl.pallas_call(
    _out_body,
    out_shape=jax.ShapeDtypeStruct((NP, 128), jnp.float32),
)


def kernel(x, edge_index, W1, b1, W2, b2):
    ei = edge_index.astype(jnp.int32)
    sp, dp = _prep(ei)
    src_p = sp.reshape(NW, S, B)
    dst_p = dp.reshape(NW, S, B)
    ones2d = jnp.ones((B, F), jnp.float32)
    zeros2d = jnp.zeros((NR, F), jnp.float32)
    w1blk = jnp.kron(jnp.eye(8, dtype=jnp.float32), W1)
    w2blk = jnp.kron(jnp.eye(8, dtype=jnp.float32), W2)
    b1t = jnp.tile(b1, 8).reshape(1, 128)
    b2t = jnp.tile(b2, 8).reshape(1, 128)
    mgrp = jnp.kron(jnp.eye(8, dtype=jnp.float32),
                    jnp.ones((F, F), jnp.float32))

    degp = _deg_pass(dst_p, ones2d, zeros2d)
    h1p = _mm1(x.reshape(NP, 8 * D), w1blk)
    dis2p, g1p = _scale(degp, h1p)
    part1 = _edge_pass(g1p.reshape(N, F), src_p, dst_p, zeros2d)
    g2p = _mid(part1, g1p, dis2p, w2blk, b1t)
    part2 = _edge_pass(g2p.reshape(N, F), src_p, dst_p, zeros2d)
    outp = _outk(part2, g2p, dis2p, b2t, mgrp)
    return outp.reshape(N, F)


# speedup vs baseline: 93.8848x; 1.0880x over previous
import functools

import jax
import jax.numpy as jnp
from jax import lax
from jax.experimental import pallas as pl
from jax.experimental.pallas import tpu as pltpu
from jax.experimental.pallas import tpu_sc as plsc

N = 10000
E = 320000
D = 128
F = 16

NC = 2
NS = 16
NW = NC * NS
EW = E // NW
B = 128
NB = 8
S = 80
G = S // NB
RPT = 640
NR = NS * RPT
SINK = NR - N
NP = N // 8
NR8 = NR // 8
RPT8 = RPT // 8
FULLR = EW // B

_mesh = plsc.VectorSubcoreMesh(
    core_axis_name="c", subcore_axis_name="s", num_cores=NC, num_subcores=NS)
_sc_params = pltpu.CompilerParams(use_tc_tiling_on_sc=False)

_IOTA = lambda: lax.iota(jnp.int32, 16)


def _build_dst_idx(ei_hbm, wid, s, d1_v, didx_v):
    pltpu.sync_copy(ei_hbm.at[1, pl.ds(wid * EW, EW)], d1_v.at[pl.ds(0, EW)])

    def bld(r, carry):
        for k in range(8):
            didx_v[r, pl.ds(k * 16, 16)] = d1_v[pl.ds(r * B + k * 16, 16)]
        return carry

    lax.fori_loop(0, FULLR, bld, 0)
    didx_v[FULLR, pl.ds(0, 16)] = d1_v[pl.ds(FULLR * B, 16)]
    for k in range(1, 8):
        didx_v[FULLR, pl.ds(k * 16, 16)] = (
            N + (s * 31 + k * 16 + _IOTA()) % SINK)
    for k in range(8):
        didx_v[FULLR + 1, pl.ds(k * 16, 16)] = (
            N + (s * 31 + 128 + k * 16 + _IOTA()) % SINK)


def _zero_acc(stage_v, acc_sh, s):
    zero = jnp.zeros((16,), jnp.float32)

    def zr(r, carry):
        stage_v[r, :] = zero
        return carry

    lax.fori_loop(0, RPT, zr, 0)
    pltpu.sync_copy(stage_v, acc_sh.at[pl.ds(s * RPT, RPT)])


def _pack_writeout(acc_sh, stage_v, pack_v, out_hbm, c, s):
    pltpu.sync_copy(acc_sh.at[pl.ds(s * RPT, RPT)], stage_v)

    def repack(r, carry):
        for k in range(8):
            pack_v[r, pl.ds(16 * k, 16)] = stage_v[r * 8 + k, :]
        return carry

    lax.fori_loop(0, RPT8, repack, 0)
    pltpu.sync_copy(pack_v, out_hbm.at[pl.ds(c * RPT8 * NS + s * RPT8, RPT8)])


@functools.partial(
    pl.kernel,
    out_type=jax.ShapeDtypeStruct((NC * NR8, 128), jnp.float32),
    mesh=_mesh,
    scratch_types=[
        pltpu.VMEM((S * B,), jnp.int32),
        pltpu.VMEM((S, B), jnp.int32),
        pltpu.VMEM((B,), jnp.float32),
        pltpu.VMEM((RPT,), jnp.float32),
        pltpu.VMEM((RPT8, 128), jnp.float32),
        pltpu.VMEM_SHARED((NR,), jnp.float32),
        pltpu.SemaphoreType.DMA,
    ],
    compiler_params=_sc_params,
)
def _deg_pass(ei_hbm, out_hbm, d1_v, didx_v, ones_v, dz_v, pack_v, deg_sh,
              sem):
    c = lax.axis_index("c")
    s = lax.axis_index("s")
    wid = s * NC + c
    one = jnp.ones((16,), jnp.float32)
    zero = jnp.zeros((16,), jnp.float32)
    for k in range(8):
        ones_v[pl.ds(k * 16, 16)] = one

    def zr(r, carry):
        dz_v[pl.ds(r * 16, 16)] = zero
        return carry

    lax.fori_loop(0, RPT // 16, zr, 0)
    pltpu.sync_copy(dz_v, deg_sh.at[pl.ds(s * RPT, RPT)])
    _build_dst_idx(ei_hbm, wid, s, d1_v, didx_v)
    plsc.subcore_barrier()

    def step(j, carry):
        pltpu.async_copy(ones_v, deg_sh.at[didx_v.at[j]], sem, add=True)
        return carry

    lax.fori_loop(0, S, step, 0)

    def drain(j, carry):
        pltpu.make_async_copy(ones_v, deg_sh.at[didx_v.at[0]], sem).wait()
        return carry

    lax.fori_loop(0, S, drain, 0)
    plsc.subcore_barrier()
    pltpu.sync_copy(deg_sh.at[pl.ds(s * RPT, RPT)], dz_v)

    def pk(rr, carry):
        v16 = dz_v[pl.ds(rr * 16, 16)]
        for k in range(8):
            pack_v[2 * rr, pl.ds(k * 16, 16)] = jnp.full(
                (16,), v16[k], jnp.float32)
            pack_v[2 * rr + 1, pl.ds(k * 16, 16)] = jnp.full(
                (16,), v16[k + 8], jnp.float32)
        return carry

    lax.fori_loop(0, RPT8 // 2, pk, 0)
    pltpu.sync_copy(pack_v, out_hbm.at[pl.ds(c * RPT8 * NS + s * RPT8, RPT8)])


@functools.partial(
    pl.kernel,
    out_type=jax.ShapeDtypeStruct((NC * NR8, 128), jnp.float32),
    mesh=_mesh,
    scratch_types=[
        pltpu.VMEM((S * B,), jnp.int32),
        pltpu.VMEM((S * B,), jnp.int32),
        pltpu.VMEM((S, B), jnp.int32),
        pltpu.VMEM((NB, B, F), jnp.float32),
        pltpu.VMEM((RPT, F), jnp.float32),
        pltpu.VMEM((RPT8, 128), jnp.float32),
        pltpu.VMEM_SHARED((NR, F), jnp.float32),
    ] + [pltpu.SemaphoreType.DMA] * (2 * NB),
    compiler_params=_sc_params,
)
def _edge_pass(g_hbm, ei_hbm, out_hbm, s1_v, d1_v, didx_v, rows_v, stage_v,
               pack_v, acc_sh, *sems):
    c = lax.axis_index("c")
    s = lax.axis_index("s")
    wid = s * NC + c
    _zero_acc(stage_v, acc_sh, s)
    pltpu.sync_copy(ei_hbm.at[0, pl.ds(wid * EW, EW)], s1_v.at[pl.ds(0, EW)])
    for k in range(EW // 16, S * B // 16):
        s1_v[pl.ds(k * 16, 16)] = (k * 16 + _IOTA()) % 128
    _build_dst_idx(ei_hbm, wid, s, d1_v, didx_v)
    plsc.subcore_barrier()

    for b in range(NB):
        pltpu.async_copy(g_hbm.at[s1_v.at[pl.ds(b * B, B)]], rows_v.at[b],
                         sems[b])

    def round_body(g, carry):
        jb = g * NB
        for b in range(NB):
            pltpu.make_async_copy(
                g_hbm.at[s1_v.at[pl.ds(0, B)]], rows_v.at[b], sems[b]).wait()
            pltpu.async_copy(
                rows_v.at[b], acc_sh.at[didx_v.at[jb + b]], sems[NB + b],
                add=True)
        for b in range(NB):
            pltpu.make_async_copy(
                rows_v.at[b], acc_sh.at[didx_v.at[0]], sems[NB + b]).wait()

            @pl.when(g < G - 1)
            def _():
                pltpu.async_copy(
                    g_hbm.at[s1_v.at[pl.ds((jb + NB + b) * B, B)]],
                    rows_v.at[b], sems[b])

        return carry

    lax.fori_loop(0, G, round_body, 0)
    plsc.subcore_barrier()
    _pack_writeout(acc_sh, stage_v, pack_v, out_hbm, c, s)


def _mm1_body(x_ref, w_ref, o_ref):
    o_ref[...] = jnp.dot(x_ref[...], w_ref[...],
                         preferred_element_type=jnp.float32)


_mm1 = pl.pallas_call(
    _mm1_body,
    out_shape=jax.ShapeDtypeStruct((NP, 128), jnp.float32),
)


def _scale_body(degp_ref, h1_ref, dis_ref, g_ref):
    deg = degp_ref[0:NP] + degp_ref[NR8:NR8 + NP] + 1.0
    dis = lax.rsqrt(deg)
    dis_ref[...] = dis
    g_ref[...] = h1_ref[...] * dis


_scale = pl.pallas_call(
    _scale_body,
    out_shape=(jax.ShapeDtypeStruct((NP, 128), jnp.float32),
               jax.ShapeDtypeStruct((NP, 128), jnp.float32)),
)


def _mid_body(part_ref, g1_ref, dis_ref, w2_ref, b1_ref, g2_ref):
    dis = dis_ref[...]
    a = (dis * (g1_ref[...] + part_ref[0:NP] + part_ref[NR8:NR8 + NP])
         + b1_ref[...])
    a = jnp.maximum(a, 0.0)
    h2 = jnp.dot(a, w2_ref[...], preferred_element_type=jnp.float32)
    g2_ref[...] = h2 * dis


_mid = pl.pallas_call(
    _mid_body,
    out_shape=jax.ShapeDtypeStruct((NP, 128), jnp.float32),
)


def _out_body(part_ref, g2_ref, dis_ref, b2_ref, mgrp_ref, o_ref):
    zp = (dis_ref[...] * (g2_ref[...] + part_ref[0:NP]
                          + part_ref[NR8:NR8 + NP]) + b2_ref[...])
    m = jnp.max(zp, axis=1, keepdims=True)
    e = jnp.exp(zp - m)
    s = jnp.dot(e, mgrp_ref[...], preferred_element_type=jnp.float32)
    o_ref[...] = zp - m - jnp.log(s)


_outk = pl.pallas_call(
    _out_body,
    out_shape=jax.ShapeDtypeStruct((NP, 128), jnp.float32),
)


def kernel(x, edge_index, W1, b1, W2, b2):
    ei = edge_index.astype(jnp.int32)
    w1blk = jnp.kron(jnp.eye(8, dtype=jnp.float32), W1)
    w2blk = jnp.kron(jnp.eye(8, dtype=jnp.float32), W2)
    b1t = jnp.tile(b1, 8).reshape(1, 128)
    b2t = jnp.tile(b2, 8).reshape(1, 128)
    mgrp = jnp.kron(jnp.eye(8, dtype=jnp.float32),
                    jnp.ones((F, F), jnp.float32))

    degp = _deg_pass(ei)
    h1p = _mm1(x.reshape(NP, 8 * D), w1blk)
    dis2p, g1p = _scale(degp, h1p)
    part1 = _edge_pass(g1p.reshape(N, F), ei)
    g2p = _mid(part1, g1p, dis2p, w2blk, b1t)
    part2 = _edge_pass(g2p.reshape(N, F), ei)
    outp = _outk(part2, g2p, dis2p, b2t, mgrp)
    return outp.reshape(N, F)


# speedup vs baseline: 98.4568x; 1.0487x over previous
import functools

import jax
import jax.numpy as jnp
from jax import lax
from jax.experimental import pallas as pl
from jax.experimental.pallas import tpu as pltpu
from jax.experimental.pallas import tpu_sc as plsc

N = 10000
E = 320000
D = 128
F = 16

NC = 2
NS = 16
NW = NC * NS
EW = E // NW
B = 128
NB = 8
S = 80
G = S // NB
RPT = 640
NR = NS * RPT
SINK = NR - N
NP = N // 8
NR8 = NR // 8
RPT8 = RPT // 8
FULLR = EW // B

_mesh = plsc.VectorSubcoreMesh(
    core_axis_name="c", subcore_axis_name="s", num_cores=NC, num_subcores=NS)
_sc_params = pltpu.CompilerParams(use_tc_tiling_on_sc=False)

_IOTA = lambda: lax.iota(jnp.int32, 16)


def _build_dst_idx(ei_hbm, wid, s, d1_v, didx_v):
    pltpu.sync_copy(ei_hbm.at[1, pl.ds(wid * EW, EW)], d1_v.at[pl.ds(0, EW)])
    _build_dst_rows(wid, s, d1_v, didx_v)


def _build_dst_rows(wid, s, d1_v, didx_v):
    def bld(r, carry):
        for k in range(8):
            didx_v[r, pl.ds(k * 16, 16)] = d1_v[pl.ds(r * B + k * 16, 16)]
        return carry

    lax.fori_loop(0, FULLR, bld, 0)
    didx_v[FULLR, pl.ds(0, 16)] = d1_v[pl.ds(FULLR * B, 16)]
    for k in range(1, 8):
        didx_v[FULLR, pl.ds(k * 16, 16)] = (
            N + (s * 31 + k * 16 + _IOTA()) % SINK)
    for k in range(8):
        didx_v[FULLR + 1, pl.ds(k * 16, 16)] = (
            N + (s * 31 + 128 + k * 16 + _IOTA()) % SINK)


def _zero_acc(stage_v, acc_sh, s):
    zero = jnp.zeros((16,), jnp.float32)

    def zr(r, carry):
        stage_v[r, :] = zero
        return carry

    lax.fori_loop(0, RPT, zr, 0)
    pltpu.sync_copy(stage_v, acc_sh.at[pl.ds(s * RPT, RPT)])


def _pack_writeout(acc_sh, stage_v, pack_v, out_hbm, c, s):
    pltpu.sync_copy(acc_sh.at[pl.ds(s * RPT, RPT)], stage_v)

    def repack(r, carry):
        for k in range(8):
            pack_v[r, pl.ds(16 * k, 16)] = stage_v[r * 8 + k, :]
        return carry

    lax.fori_loop(0, RPT8, repack, 0)
    pltpu.sync_copy(pack_v, out_hbm.at[pl.ds(c * RPT8 * NS + s * RPT8, RPT8)])


@functools.partial(
    pl.kernel,
    out_type=jax.ShapeDtypeStruct((NC * NR8, 128), jnp.float32),
    mesh=_mesh,
    scratch_types=[
        pltpu.VMEM((S * B,), jnp.int32),
        pltpu.VMEM((S, B), jnp.int32),
        pltpu.VMEM((B,), jnp.float32),
        pltpu.VMEM((RPT,), jnp.float32),
        pltpu.VMEM((RPT8, 128), jnp.float32),
        pltpu.VMEM_SHARED((NR,), jnp.float32),
        pltpu.SemaphoreType.DMA,
    ],
    compiler_params=_sc_params,
)
def _deg_pass(ei_hbm, out_hbm, d1_v, didx_v, ones_v, dz_v, pack_v, deg_sh,
              sem):
    c = lax.axis_index("c")
    s = lax.axis_index("s")
    wid = s * NC + c
    one = jnp.ones((16,), jnp.float32)
    zero = jnp.zeros((16,), jnp.float32)
    for k in range(8):
        ones_v[pl.ds(k * 16, 16)] = one

    def zr(r, carry):
        dz_v[pl.ds(r * 16, 16)] = zero
        return carry

    lax.fori_loop(0, RPT // 16, zr, 0)
    pltpu.sync_copy(dz_v, deg_sh.at[pl.ds(s * RPT, RPT)])
    _build_dst_idx(ei_hbm, wid, s, d1_v, didx_v)
    plsc.subcore_barrier()

    def step(j, carry):
        pltpu.async_copy(ones_v, deg_sh.at[didx_v.at[j]], sem, add=True)
        return carry

    lax.fori_loop(0, S, step, 0)

    def drain(j, carry):
        pltpu.make_async_copy(ones_v, deg_sh.at[didx_v.at[0]], sem).wait()
        return carry

    lax.fori_loop(0, S, drain, 0)
    plsc.subcore_barrier()
    pltpu.sync_copy(deg_sh.at[pl.ds(s * RPT, RPT)], dz_v)

    def pk(rr, carry):
        v16 = dz_v[pl.ds(rr * 16, 16)]
        for k in range(8):
            pack_v[2 * rr, pl.ds(k * 16, 16)] = jnp.full(
                (16,), v16[k], jnp.float32)
            pack_v[2 * rr + 1, pl.ds(k * 16, 16)] = jnp.full(
                (16,), v16[k + 8], jnp.float32)
        return carry

    lax.fori_loop(0, RPT8 // 2, pk, 0)
    pltpu.sync_copy(pack_v, out_hbm.at[pl.ds(c * RPT8 * NS + s * RPT8, RPT8)])


@functools.partial(
    pl.kernel,
    out_type=jax.ShapeDtypeStruct((NC * NR8, 128), jnp.float32),
    mesh=_mesh,
    scratch_types=[
        pltpu.VMEM((S * B,), jnp.int32),
        pltpu.VMEM((S * B,), jnp.int32),
        pltpu.VMEM((S, B), jnp.int32),
        pltpu.VMEM((NB, B, F), jnp.float32),
        pltpu.VMEM((RPT, F), jnp.float32),
        pltpu.VMEM((RPT8, 128), jnp.float32),
        pltpu.VMEM_SHARED((NR, F), jnp.float32),
    ] + [pltpu.SemaphoreType.DMA] * (2 * NB + 2),
    compiler_params=_sc_params,
)
def _edge_pass(g_hbm, ei_hbm, out_hbm, s1_v, d1_v, didx_v, rows_v, stage_v,
               pack_v, acc_sh, *sems):
    c = lax.axis_index("c")
    s = lax.axis_index("s")
    wid = s * NC + c
    cp_s = pltpu.async_copy(ei_hbm.at[0, pl.ds(wid * EW, EW)],
                            s1_v.at[pl.ds(0, EW)], sems[2 * NB])
    cp_d = pltpu.async_copy(ei_hbm.at[1, pl.ds(wid * EW, EW)],
                            d1_v.at[pl.ds(0, EW)], sems[2 * NB + 1])
    _zero_acc(stage_v, acc_sh, s)
    cp_s.wait()
    cp_d.wait()
    for k in range(EW // 16, S * B // 16):
        s1_v[pl.ds(k * 16, 16)] = (k * 16 + _IOTA()) % 128
    _build_dst_rows(wid, s, d1_v, didx_v)
    plsc.subcore_barrier()

    for b in range(NB):
        pltpu.async_copy(g_hbm.at[s1_v.at[pl.ds(b * B, B)]], rows_v.at[b],
                         sems[b])

    def round_body(g, carry):
        jb = g * NB
        for b in range(NB):
            pltpu.make_async_copy(
                g_hbm.at[s1_v.at[pl.ds(0, B)]], rows_v.at[b], sems[b]).wait()
            pltpu.async_copy(
                rows_v.at[b], acc_sh.at[didx_v.at[jb + b]], sems[NB + b],
                add=True)
        for b in range(NB):
            pltpu.make_async_copy(
                rows_v.at[b], acc_sh.at[didx_v.at[0]], sems[NB + b]).wait()

            @pl.when(g < G - 1)
            def _():
                pltpu.async_copy(
                    g_hbm.at[s1_v.at[pl.ds((jb + NB + b) * B, B)]],
                    rows_v.at[b], sems[b])

        return carry

    lax.fori_loop(0, G, round_body, 0)
    plsc.subcore_barrier()
    _pack_writeout(acc_sh, stage_v, pack_v, out_hbm, c, s)


def _mm1_body(x_ref, w_ref, o_ref):
    o_ref[...] = jnp.dot(x_ref[...], w_ref[...],
                         preferred_element_type=jnp.float32)


_mm1 = pl.pallas_call(
    _mm1_body,
    out_shape=jax.ShapeDtypeStruct((NP, 128), jnp.float32),
)


def _scale_body(degp_ref, h1_ref, dis_ref, g_ref):
    deg = degp_ref[0:NP] + degp_ref[NR8:NR8 + NP] + 1.0
    dis = lax.rsqrt(deg)
    dis_ref[...] = dis
    g_ref[...] = h1_ref[...] * dis


_scale = pl.pallas_call(
    _scale_body,
    out_shape=(jax.ShapeDtypeStruct((NP, 128), jnp.float32),
               jax.ShapeDtypeStruct((NP, 128), jnp.float32)),
)


def _mid_body(part_ref, g1_ref, dis_ref, w2_ref, b1_ref, g2_ref):
    dis = dis_ref[...]
    a = (dis * (g1_ref[...] + part_ref[0:NP] + part_ref[NR8:NR8 + NP])
         + b1_ref[...])
    a = jnp.maximum(a, 0.0)
    h2 = jnp.dot(a, w2_ref[...], preferred_element_type=jnp.float32)
    g2_ref[...] = h2 * dis


_mid = pl.pallas_call(
    _mid_body,
    out_shape=jax.ShapeDtypeStruct((NP, 128), jnp.float32),
)


def _out_body(part_ref, g2_ref, dis_ref, b2_ref, mgrp_ref, o_ref):
    zp = (dis_ref[...] * (g2_ref[...] + part_ref[0:NP]
                          + part_ref[NR8:NR8 + NP]) + b2_ref[...])
    m = jnp.max(zp, axis=1, keepdims=True)
    e = jnp.exp(zp - m)
    s = jnp.dot(e, mgrp_ref[...], preferred_element_type=jnp.float32)
    o_ref[...] = zp - m - jnp.log(s)


_outk = pl.pallas_call(
    _out_body,
    out_shape=jax.ShapeDtypeStruct((NP, 128), jnp.float32),
)


def kernel(x, edge_index, W1, b1, W2, b2):
    ei = edge_index.astype(jnp.int32)
    w1blk = jnp.kron(jnp.eye(8, dtype=jnp.float32), W1)
    w2blk = jnp.kron(jnp.eye(8, dtype=jnp.float32), W2)
    b1t = jnp.tile(b1, 8).reshape(1, 128)
    b2t = jnp.tile(b2, 8).reshape(1, 128)
    mgrp = jnp.kron(jnp.eye(8, dtype=jnp.float32),
                    jnp.ones((F, F), jnp.float32))

    degp = _deg_pass(ei)
    h1p = _mm1(x.reshape(NP, 8 * D), w1blk)
    dis2p, g1p = _scale(degp, h1p)
    part1 = _edge_pass(g1p.reshape(N, F), ei)
    g2p = _mid(part1, g1p, dis2p, w2blk, b1t)
    part2 = _edge_pass(g2p.reshape(N, F), ei)
    outp = _outk(part2, g2p, dis2p, b2t, mgrp)
    return outp.reshape(N, F)
